# baseline (speedup 1.0000x reference)
import jax
import jax.numpy as jnp
from jax.experimental import pallas as pl

B = 16384
N_CAT = 26
N_DENSE = 13
NUM_BINS = 128
CROSS_PAIRS = ((0, 1), (2, 3), (4, 5))
OUT_W = N_CAT * NUM_BINS + len(CROSS_PAIRS) * NUM_BINS + N_DENSE
BLK = 256


def _body(x_ref, f_ref, o_ref):
    x = x_ref[...]
    col = jax.lax.broadcasted_iota(jnp.int32, (BLK, NUM_BINS), 1)
    for f in range(N_CAT):
        h = (x[:, f : f + 1] * 31 + 17) & 127
        o_ref[:, f * NUM_BINS : (f + 1) * NUM_BINS] = (h == col).astype(jnp.float32)
    base = N_CAT * NUM_BINS
    for k, (i, j) in enumerate(CROSS_PAIRS):
        a = x[:, i : i + 1] % 32749
        b = x[:, j : j + 1] % 32749
        comb = a * 32749 + b
        hc = (comb * 31 + 17) & 127
        o_ref[:, base + k * NUM_BINS : base + (k + 1) * NUM_BINS] = (hc == col).astype(
            jnp.float32
        )
    o_ref[:, base + len(CROSS_PAIRS) * NUM_BINS :] = f_ref[...]


def kernel(int_features, float_features):
    return pl.pallas_call(
        _body,
        grid=(B // BLK,),
        in_specs=[
            pl.BlockSpec((BLK, N_CAT), lambda i: (i, 0)),
            pl.BlockSpec((BLK, N_DENSE), lambda i: (i, 0)),
        ],
        out_specs=pl.BlockSpec((BLK, OUT_W), lambda i: (i, 0)),
        out_shape=jax.ShapeDtypeStruct((B, OUT_W), jnp.float32),
    )(int_features, float_features)
